# Initial kernel scaffold; baseline (speedup 1.0000x reference)
import jax
import jax.numpy as jnp
from jax.experimental import pallas as pl


def kernel(x, Wg, bg, W, b):
    raise NotImplementedError("write your pallas kernel here")


# speedup vs baseline: 2.0387x; 2.0387x over previous
import functools

import jax
import jax.numpy as jnp
from jax import lax
from jax.experimental import pallas as pl
from jax.experimental.pallas import tpu as pltpu

T = 2048
D = 1024
E = 8
TOP_K = 2


def _moe_dense_body(x_ref, wg_ref, bg_ref, w_ref, b_ref,
                    out_ref, idx_ref, comb_ref):
    e = pl.program_id(0)

    @pl.when(e == 0)
    def _gate():
        logits = jnp.dot(x_ref[...], wg_ref[...],
                         preferred_element_type=jnp.float32) + bg_ref[...]
        col = lax.broadcasted_iota(jnp.int32, (T, E), 1)
        m1 = jnp.max(logits, axis=1, keepdims=True)
        i1 = jnp.min(jnp.where(logits == m1, col, E), axis=1, keepdims=True)
        masked = jnp.where(col == i1, -jnp.inf, logits)
        m2 = jnp.max(masked, axis=1, keepdims=True)
        i2 = jnp.min(jnp.where(masked == m2, col, E), axis=1, keepdims=True)
        r = jnp.exp(m2 - m1)
        w1 = 1.0 / (1.0 + r)
        w2 = r / (1.0 + r)
        comb_ref[...] = (jnp.where(col == i1, w1, 0.0)
                         + jnp.where(col == i2, w2, 0.0))
        idx_ref[...] = jnp.concatenate([i1, i2], axis=1)

    ce = jnp.sum(
        comb_ref[...] * (lax.broadcasted_iota(jnp.int32, (T, E), 1) == e),
        axis=1, keepdims=True)
    acc = jnp.dot(x_ref[...], w_ref[0], preferred_element_type=jnp.float32)
    contrib = ce * (acc + b_ref[0])

    @pl.when(e == 0)
    def _init():
        out_ref[...] = contrib

    @pl.when(e > 0)
    def _acc():
        out_ref[...] += contrib


@jax.jit
def kernel(x, Wg, bg, W, b):
    bg2 = bg.reshape(1, E)
    b3 = b.reshape(E, 1, D)
    grid = (E,)
    out, idx = pl.pallas_call(
        _moe_dense_body,
        grid=grid,
        in_specs=[
            pl.BlockSpec((T, D), lambda e: (0, 0)),
            pl.BlockSpec((D, E), lambda e: (0, 0)),
            pl.BlockSpec((1, E), lambda e: (0, 0)),
            pl.BlockSpec((1, D, D), lambda e: (e, 0, 0)),
            pl.BlockSpec((1, 1, D), lambda e: (e, 0, 0)),
        ],
        out_specs=[
            pl.BlockSpec((T, D), lambda e: (0, 0)),
            pl.BlockSpec((T, TOP_K), lambda e: (0, 0)),
        ],
        out_shape=[
            jax.ShapeDtypeStruct((T, D), jnp.float32),
            jax.ShapeDtypeStruct((T, TOP_K), jnp.int32),
        ],
        scratch_shapes=[pltpu.VMEM((T, E), jnp.float32)],
        compiler_params=pltpu.CompilerParams(
            dimension_semantics=("arbitrary",),
        ),
    )(x, Wg, bg2, W, b3)
    return out, idx
